# baseline (speedup 1.0000x reference)
import functools

import jax
import jax.numpy as jnp
from jax import lax
from jax.experimental import pallas as pl
from jax.experimental.pallas import tpu as pltpu
from jax.experimental.pallas import tpu_sc as plsc

_NC = 2
_NS = 16
_NW = _NC * _NS

_BATCH = 16384
_EMB = 16


def _sc_gather_body(sess_ids, item_ids, sg_t, ig_t, sm_t, im_t,
                    out_sg, out_ig, out_sm, out_im,
                    sidx, iidx, bsg, big, bsm, bim,
                    sem0, sem1, sem2, sem3):
  bpw = _BATCH // _NW
  wid = lax.axis_index("s") * _NC + lax.axis_index("c")
  base = wid * bpw
  pltpu.sync_copy(sess_ids.at[pl.ds(base, bpw)], sidx)
  pltpu.sync_copy(item_ids.at[pl.ds(base, bpw)], iidx)
  c0 = pltpu.async_copy(sg_t.at[sidx], bsg, sem0)
  c1 = pltpu.async_copy(ig_t.at[iidx], big, sem1)
  c2 = pltpu.async_copy(sm_t.at[sidx], bsm, sem2)
  c3 = pltpu.async_copy(im_t.at[iidx], bim, sem3)
  c0.wait()
  c1.wait()
  c2.wait()
  c3.wait()
  pltpu.sync_copy(bsg, out_sg.at[pl.ds(base, bpw)])
  pltpu.sync_copy(big, out_ig.at[pl.ds(base, bpw)])
  pltpu.sync_copy(bsm, out_sm.at[pl.ds(base, bpw)])
  pltpu.sync_copy(bim, out_im.at[pl.ds(base, bpw)])


def _sc_gather(sess_ids, item_ids, sg_t, ig_t, sm_t, im_t):
  bpw = _BATCH // _NW
  mesh = plsc.VectorSubcoreMesh(core_axis_name="c", subcore_axis_name="s",
                                num_cores=_NC, num_subcores=_NS)
  row = jax.ShapeDtypeStruct((_BATCH, _EMB), jnp.float32)
  f = pl.kernel(
      _sc_gather_body,
      out_type=[row, row, row, row],
      mesh=mesh,
      scratch_types=[
          pltpu.VMEM((bpw,), jnp.int32),
          pltpu.VMEM((bpw,), jnp.int32),
          pltpu.VMEM((bpw, _EMB), jnp.float32),
          pltpu.VMEM((bpw, _EMB), jnp.float32),
          pltpu.VMEM((bpw, _EMB), jnp.float32),
          pltpu.VMEM((bpw, _EMB), jnp.float32),
          pltpu.SemaphoreType.DMA,
          pltpu.SemaphoreType.DMA,
          pltpu.SemaphoreType.DMA,
          pltpu.SemaphoreType.DMA,
      ],
      compiler_params=pltpu.CompilerParams(use_tc_tiling_on_sc=False),
  )
  return f(sess_ids, item_ids, sg_t, ig_t, sm_t, im_t)


def _tc_mlp_body(sg, ig, sm, im, w1, b1, w2, b2, w3, b3, wo, bo, out):
  dn = (((1,), (1,)), ((), ()))
  smv = sm[...]
  imv = im[...]
  w1v = w1[...]
  h = lax.dot_general(smv, w1v[:, :_EMB], dn,
                      preferred_element_type=jnp.float32)
  h += lax.dot_general(imv, w1v[:, _EMB:], dn,
                       preferred_element_type=jnp.float32)
  h = jnp.maximum(h + b1[...], 0.0)
  h = lax.dot_general(h, w2[...], dn, preferred_element_type=jnp.float32)
  h = jnp.maximum(h + b2[...], 0.0)
  h = lax.dot_general(h, w3[...], dn, preferred_element_type=jnp.float32)
  h = jnp.maximum(h + b3[...], 0.0)
  gmf = sg[...] * ig[...]
  wov = wo[...]
  logit = lax.dot_general(gmf, wov[:, :_EMB], dn,
                          preferred_element_type=jnp.float32)
  logit += lax.dot_general(h, wov[:, _EMB:], dn,
                           preferred_element_type=jnp.float32)
  out[...] = jax.nn.sigmoid(logit + bo[...])


def _tc_mlp(sg, ig, sm, im, w1, b1, w2, b2, w3, b3, wo, bo):
  chunk = 2048
  grid = (_BATCH // chunk,)
  emb_spec = pl.BlockSpec((chunk, _EMB), lambda i: (i, 0))

  def full(shape):
    return pl.BlockSpec(shape, lambda i: tuple(0 for _ in shape))

  out = pl.pallas_call(
      _tc_mlp_body,
      grid=grid,
      in_specs=[
          emb_spec, emb_spec, emb_spec, emb_spec,
          full((64, 32)), full((64,)),
          full((32, 64)), full((32,)),
          full((16, 32)), full((16,)),
          full((1, 32)), full((1,)),
      ],
      out_specs=pl.BlockSpec((chunk, 1), lambda i: (i, 0)),
      out_shape=jax.ShapeDtypeStruct((_BATCH, 1), jnp.float32),
  )(sg, ig, sm, im, w1, b1, w2, b2, w3, b3, wo, bo)
  return jnp.squeeze(out, axis=-1)


@jax.jit
def kernel(sess_ids, item_ids, sess_emb_gmf, item_emb_gmf, sess_emb_mlp,
           item_emb_mlp, W1, b1, W2, b2, W3, b3, Wo, bo):
  sg, ig, sm, im = _sc_gather(sess_ids, item_ids, sess_emb_gmf, item_emb_gmf,
                              sess_emb_mlp, item_emb_mlp)
  return _tc_mlp(sg, ig, sm, im, W1, b1, W2, b2, W3, b3, Wo, bo)
